# baseline (speedup 1.0000x reference)
import jax
import jax.numpy as jnp
from jax.experimental import pallas as pl

_SELU_SCALE = 1.0507009873554805
_SELU_ALPHA = 1.6732632423543772

_BLOCK_B = 128


def _fused_kernel(emb_ref, cat_ref, prem_ref, sent_ref, temp_ref, week_ref,
                  hour_ref, w1_ref, b1_ref, w2_ref, b2_ref,
                  ptab_ref, stab_ref, ttab_ref, wtab_ref, htab_ref, out_ref):
    RB, L, ART = emb_ref.shape
    lookups = ((prem_ref, ptab_ref), (sent_ref, stab_ref),
               (temp_ref, ttab_ref), (week_ref, wtab_ref),
               (hour_ref, htab_ref))
    w1a = w1_ref[:ART, :]
    w1b = w1_ref[ART:, :]
    for l in range(L):
        h = jnp.dot(emb_ref[:, l, :], w1a, preferred_element_type=jnp.float32)
        h += jnp.dot(cat_ref[:, l, :], w1b, preferred_element_type=jnp.float32)
        h += b1_ref[...]
        h = _SELU_SCALE * jnp.where(h > 0, h, _SELU_ALPHA * (jnp.exp(h) - 1.0))
        x = jnp.dot(h, w2_ref[...], preferred_element_type=jnp.float32)
        x += b2_ref[...]
        for idx_ref, tab_ref in lookups:
            k = tab_ref.shape[0]
            iota = jax.lax.broadcasted_iota(jnp.int32, (1, k), 1)
            oh = (idx_ref[:, l:l + 1] == iota).astype(jnp.float32)
            x += jnp.dot(oh, tab_ref[...], preferred_element_type=jnp.float32)
        out_ref[:, l, :] = x


def kernel(embs, cat_embs, premium, sentiment, mask, temporal, weekdays, hours,
           W1, b1, W2, b2, premium_tab, sentiment_tab, temporal_tab,
           weekday_tab, hour_tab):
    B, L, ART = embs.shape
    CAT = cat_embs.shape[2]
    DIMS = W2.shape[1]
    RB = _BLOCK_B
    grid = B // RB

    idx_spec = pl.BlockSpec((RB, L), lambda i: (i, 0))

    def tab_spec(t):
        return pl.BlockSpec(t.shape, lambda i: (0, 0))

    out = pl.pallas_call(
        _fused_kernel,
        grid=(grid,),
        in_specs=[
            pl.BlockSpec((RB, L, ART), lambda i: (i, 0, 0)),
            pl.BlockSpec((RB, L, CAT), lambda i: (i, 0, 0)),
            idx_spec, idx_spec, idx_spec, idx_spec, idx_spec,
            pl.BlockSpec((ART + CAT, DIMS), lambda i: (0, 0)),
            pl.BlockSpec((1, DIMS), lambda i: (0, 0)),
            pl.BlockSpec((DIMS, DIMS), lambda i: (0, 0)),
            pl.BlockSpec((1, DIMS), lambda i: (0, 0)),
            tab_spec(premium_tab), tab_spec(sentiment_tab),
            tab_spec(temporal_tab), tab_spec(weekday_tab), tab_spec(hour_tab),
        ],
        out_specs=pl.BlockSpec((RB, L, DIMS), lambda i: (i, 0, 0)),
        out_shape=jax.ShapeDtypeStruct((B, L, DIMS), jnp.float32),
    )(embs, cat_embs, premium, sentiment, temporal, weekdays, hours,
      W1, b1.reshape(1, DIMS), W2, b2.reshape(1, DIMS),
      premium_tab, sentiment_tab, temporal_tab, weekday_tab, hour_tab)

    return (out, mask)
